# baseline (speedup 1.0000x reference)
import jax
import jax.numpy as jnp
from jax.experimental import pallas as pl

W = 256
H = 16
M = 144
ROWS = 529
IB = 8
NB = 1


def _gather_body(rpi_ref, btT_ref, out_ref):
    iota = jax.lax.broadcasted_iota(jnp.int32, (ROWS, M), 0)
    btT = btT_ref[...]
    for rr in range(IB):
        onehot = (rpi_ref[rr:rr + 1, :] == iota).astype(jnp.float32)
        out_ref[:, rr, :] = jnp.dot(btT, onehot,
                                    preferred_element_type=jnp.float32)


def _add_body(att_ref, bias_ref, out_ref):
    out_ref[...] = att_ref[...] + bias_ref[...][None]


def kernel(att_scores, bias_table, relative_position_index):
    bias = pl.pallas_call(
        _gather_body,
        grid=(M // IB,),
        in_specs=[
            pl.BlockSpec((IB, M), lambda c: (c, 0)),
            pl.BlockSpec((H, ROWS), lambda c: (0, 0)),
        ],
        out_specs=pl.BlockSpec((H, IB, M), lambda c: (0, c, 0)),
        out_shape=jax.ShapeDtypeStruct((H, M, M), jnp.float32),
    )(relative_position_index, bias_table.T)

    return pl.pallas_call(
        _add_body,
        grid=(W // NB,),
        in_specs=[
            pl.BlockSpec((NB, H, M, M), lambda w: (w, 0, 0, 0)),
            pl.BlockSpec((H, M, M), lambda w: (0, 0, 0)),
        ],
        out_specs=pl.BlockSpec((NB, H, M, M), lambda w: (w, 0, 0, 0)),
        out_shape=jax.ShapeDtypeStruct((W, H, M, M), jnp.float32),
    )(att_scores, bias)


# speedup vs baseline: 1.0180x; 1.0180x over previous
import jax
import jax.numpy as jnp
from jax.experimental import pallas as pl
from jax.experimental.pallas import tpu as pltpu

W = 256
H = 16
M = 144
ROWS = 529
IB = 8
NBUF = 8


def _gather_body(rpi_ref, btT_ref, out_ref):
    iota = jax.lax.broadcasted_iota(jnp.int32, (ROWS, M), 0)
    btT = btT_ref[...]
    for rr in range(IB):
        onehot = (rpi_ref[rr:rr + 1, :] == iota).astype(jnp.float32)
        out_ref[:, rr, :] = jnp.dot(btT, onehot,
                                    preferred_element_type=jnp.float32)


def _add_body(att_hbm, bias_ref, out_hbm, in_bufs, out_bufs, in_sems, out_sems):
    g = pl.program_id(0)
    slot = jax.lax.rem(g, NBUF)

    @pl.when(g == 0)
    def _prologue():
        for k in range(NBUF):
            pltpu.make_async_copy(att_hbm.at[k], in_bufs.at[k],
                                  in_sems.at[k]).start()

    pltpu.make_async_copy(att_hbm.at[g], in_bufs.at[slot],
                          in_sems.at[slot]).wait()

    @pl.when(g >= NBUF)
    def _drain_out():
        pltpu.make_async_copy(out_bufs.at[slot], out_hbm.at[g - NBUF],
                              out_sems.at[slot]).wait()

    out_bufs[slot] = in_bufs[slot] + bias_ref[...]
    pltpu.make_async_copy(out_bufs.at[slot], out_hbm.at[g],
                          out_sems.at[slot]).start()

    @pl.when(g + NBUF < W)
    def _refill():
        pltpu.make_async_copy(att_hbm.at[g + NBUF], in_bufs.at[slot],
                              in_sems.at[slot]).start()

    @pl.when(g == W - 1)
    def _epilogue():
        for k in range(NBUF):
            pltpu.make_async_copy(out_bufs.at[k], out_hbm.at[W - NBUF + k],
                                  out_sems.at[k]).wait()


def kernel(att_scores, bias_table, relative_position_index):
    bias = pl.pallas_call(
        _gather_body,
        grid=(M // IB,),
        in_specs=[
            pl.BlockSpec((IB, M), lambda c: (c, 0)),
            pl.BlockSpec((H, ROWS), lambda c: (0, 0)),
        ],
        out_specs=pl.BlockSpec((H, IB, M), lambda c: (0, c, 0)),
        out_shape=jax.ShapeDtypeStruct((H, M, M), jnp.float32),
    )(relative_position_index, bias_table.T)

    return pl.pallas_call(
        _add_body,
        grid=(W,),
        in_specs=[
            pl.BlockSpec(memory_space=pl.ANY),
            pl.BlockSpec((H, M, M), lambda w: (0, 0, 0)),
        ],
        out_specs=pl.BlockSpec(memory_space=pl.ANY),
        out_shape=jax.ShapeDtypeStruct((W, H, M, M), jnp.float32),
        scratch_shapes=[
            pltpu.VMEM((NBUF, H, M, M), jnp.float32),
            pltpu.VMEM((NBUF, H, M, M), jnp.float32),
            pltpu.SemaphoreType.DMA((NBUF,)),
            pltpu.SemaphoreType.DMA((NBUF,)),
        ],
    )(att_scores, bias)


# speedup vs baseline: 1.7933x; 1.7615x over previous
import jax
import jax.numpy as jnp
from jax.experimental import pallas as pl
from jax.experimental.pallas import tpu as pltpu

W = 256
H = 16
M = 144
ROWS = 529
IB = 8
NB = 4
SL = H * M * M // 128


def _gather_body(rpi_ref, btT_ref, out_ref):
    iota = jax.lax.broadcasted_iota(jnp.int32, (ROWS, M), 0)
    btT = btT_ref[...]
    for rr in range(IB):
        onehot = (rpi_ref[rr:rr + 1, :] == iota).astype(jnp.float32)
        out_ref[:, rr, :] = jnp.dot(btT, onehot,
                                    preferred_element_type=jnp.float32)


def _add_body(att_ref, bias_ref, out_ref):
    out_ref[...] = att_ref[...] + bias_ref[...][None]


def kernel(att_scores, bias_table, relative_position_index):
    bias = pl.pallas_call(
        _gather_body,
        grid=(M // IB,),
        in_specs=[
            pl.BlockSpec((IB, M), lambda c: (c, 0)),
            pl.BlockSpec((H, ROWS), lambda c: (0, 0)),
        ],
        out_specs=pl.BlockSpec((H, IB, M), lambda c: (0, c, 0)),
        out_shape=jax.ShapeDtypeStruct((H, M, M), jnp.float32),
    )(relative_position_index, bias_table.T)

    att3 = att_scores.reshape(W, SL, 128)
    bias2 = bias.reshape(SL, 128)
    out3 = pl.pallas_call(
        _add_body,
        grid=(W // NB,),
        in_specs=[
            pl.BlockSpec((NB, SL, 128), lambda w: (w, 0, 0)),
            pl.BlockSpec((SL, 128), lambda w: (0, 0)),
        ],
        out_specs=pl.BlockSpec((NB, SL, 128), lambda w: (w, 0, 0)),
        out_shape=jax.ShapeDtypeStruct((W, SL, 128), jnp.float32),
    )(att3, bias2)
    return out3.reshape(W, H, M, M)
